# baseline (speedup 1.0000x reference)
import functools

import jax
import jax.numpy as jnp
from jax import lax
from jax.experimental import pallas as pl
from jax.experimental.pallas import tpu as pltpu
from jax.experimental.pallas import tpu_sc as plsc

_N_CLASSES = 1000000
_V = _N_CLASSES + 1
_N_CHANNELS = 32
_BATCH = 16384
_DROP_P = 0.1

_NC = 2
_NS = 16
_NW = _NC * _NS
_BPW = _BATCH // _NW
_RND = 16
_NROUND = _BPW // _RND

_mesh = plsc.VectorSubcoreMesh(core_axis_name="c", subcore_axis_name="s")


@functools.partial(
    pl.kernel,
    mesh=_mesh,
    out_type=jax.ShapeDtypeStruct((_N_CHANNELS, _BATCH), jnp.float32),
    scratch_types=[
        pltpu.VMEM((_BPW,), jnp.int32),
        pltpu.VMEM((_N_CHANNELS,), jnp.int32),
        pltpu.VMEM((_RND, _N_CHANNELS, 128), jnp.float32),
        pltpu.VMEM((_N_CHANNELS, _BPW), jnp.float32),
        pltpu.SemaphoreType.DMA,
    ],
    compiler_params=pltpu.CompilerParams(needs_layout_passes=False),
)
def _sc_lookup(tbl_hbm, lab_hbm, out_hbm, idx_v, ch_v, stage_v, cols_v, sem):
    wid = lax.axis_index("s") * _NC + lax.axis_index("c")
    base = wid * _BPW
    pltpu.sync_copy(lab_hbm.at[pl.ds(base, _BPW)], idx_v)
    ch_v[pl.ds(0, 16)] = lax.iota(jnp.int32, 16)
    ch_v[pl.ds(16, 16)] = lax.iota(jnp.int32, 16) + jnp.int32(16)

    def round_body(r, _):
        labs = idx_v[pl.ds(r * _RND, _RND)]
        c0s = labs & jnp.int32(-128)
        for u in range(_RND):
            pltpu.async_copy(
                tbl_hbm.at[ch_v, pl.ds(pl.multiple_of(c0s[u], 128), 128)],
                stage_v.at[u],
                sem,
            )
        for u in range(_RND):
            pltpu.make_async_copy(
                tbl_hbm.at[ch_v, pl.ds(0, 128)], stage_v.at[u], sem
            ).wait()

        cvec = labs & jnp.int32(127)
        kvec = lax.iota(jnp.int32, 16)
        for j in range(_N_CHANNELS):
            jvec = jnp.full((16,), j, jnp.int32)
            val = plsc.load_gather(stage_v, [kvec, jvec, cvec])
            cols_v[j, pl.ds(r * _RND, _RND)] = val
        return 0

    lax.fori_loop(0, _NROUND, round_body, 0)
    pltpu.sync_copy(cols_v, out_hbm.at[:, pl.ds(base, _BPW)])


def kernel(labels, if_train, embedding_table):
    def _masked(lab):
        drop_key = jax.random.key(1)
        drop = jax.random.uniform(drop_key, (lab.shape[0],)) < _DROP_P
        return jnp.where(drop, jnp.int32(_N_CLASSES), lab)

    lab = lax.cond(jnp.asarray(if_train) != 0, _masked, lambda l: l, labels)
    out_t = _sc_lookup(embedding_table.T, lab)
    return out_t.T


# speedup vs baseline: 1.0847x; 1.0847x over previous
import functools

import jax
import jax.numpy as jnp
from jax import lax
from jax.experimental import pallas as pl
from jax.experimental.pallas import tpu as pltpu
from jax.experimental.pallas import tpu_sc as plsc

_N_CLASSES = 1000000
_V = _N_CLASSES + 1
_N_CHANNELS = 32
_BATCH = 16384
_DROP_P = 0.1

_NC = 2
_NS = 16
_NW = _NC * _NS
_BPW = _BATCH // _NW
_RND = 16
_NROUND = _BPW // _RND

_mesh = plsc.VectorSubcoreMesh(core_axis_name="c", subcore_axis_name="s")


@functools.partial(
    pl.kernel,
    mesh=_mesh,
    out_type=jax.ShapeDtypeStruct((_N_CHANNELS, _BATCH), jnp.float32),
    scratch_types=[
        pltpu.VMEM((_BPW,), jnp.int32),
        pltpu.VMEM((_RND, 4, 8, 128), jnp.float32),
        pltpu.VMEM((_N_CHANNELS, _BPW), jnp.float32),
        pltpu.SemaphoreType.DMA,
    ],
    compiler_params=pltpu.CompilerParams(needs_layout_passes=False),
)
def _sc_lookup(tbl3_hbm, lab_hbm, out_hbm, idx_v, stage_v, cols_v, sem):
    wid = lax.axis_index("s") * _NC + lax.axis_index("c")
    base = wid * _BPW
    pltpu.sync_copy(lab_hbm.at[pl.ds(base, _BPW)], idx_v)

    def round_body(r, _):
        labs = idx_v[pl.ds(r * _RND, _RND)]
        c0s = labs & jnp.int32(-128)
        for u in range(_RND):
            pltpu.async_copy(
                tbl3_hbm.at[:, :, pl.ds(pl.multiple_of(c0s[u], 128), 128)],
                stage_v.at[u],
                sem,
            )
        for u in range(_RND):
            pltpu.make_async_copy(
                tbl3_hbm.at[:, :, pl.ds(0, 128)], stage_v.at[u], sem
            ).wait()

        cvec = labs & jnp.int32(127)
        kvec = lax.iota(jnp.int32, 16)
        for j in range(_N_CHANNELS):
            g = jnp.full((16,), j // 8, jnp.int32)
            s = jnp.full((16,), j % 8, jnp.int32)
            val = plsc.load_gather(stage_v, [kvec, g, s, cvec])
            cols_v[j, pl.ds(r * _RND, _RND)] = val
        return 0

    lax.fori_loop(0, _NROUND, round_body, 0)
    pltpu.sync_copy(cols_v, out_hbm.at[:, pl.ds(base, _BPW)])


def kernel(labels, if_train, embedding_table):
    def _masked(lab):
        drop_key = jax.random.key(1)
        drop = jax.random.uniform(drop_key, (lab.shape[0],)) < _DROP_P
        return jnp.where(drop, jnp.int32(_N_CLASSES), lab)

    lab = lax.cond(jnp.asarray(if_train) != 0, _masked, lambda l: l, labels)
    tbl3 = embedding_table.T.reshape(4, 8, _V)
    out_t = _sc_lookup(tbl3, lab)
    return out_t.T
